# Initial kernel scaffold; baseline (speedup 1.0000x reference)
import jax
import jax.numpy as jnp
from jax.experimental import pallas as pl


def kernel(outputs, lables, masks, weight):
    raise NotImplementedError("write your pallas kernel here")


# speedup vs baseline: 4.9271x; 4.9271x over previous
import functools

import jax
import jax.numpy as jnp
from jax import lax
from jax.experimental import pallas as pl
from jax.experimental.pallas import tpu as pltpu
from jax.experimental.pallas import tpu_sc as plsc

NC = 2
NS = 16
L = 16

LN2 = 0.6931471805599453
SQRT2 = 1.4142135623730951


def _log_1_to_k(s):
    bits = plsc.bitcast(s, jnp.int32)
    e = (bits >> 23) - 127
    mbits = (bits & jnp.int32(0x007FFFFF)) | jnp.int32(0x3F800000)
    m = plsc.bitcast(mbits, jnp.float32)
    big = m > jnp.float32(SQRT2)
    m = jnp.where(big, m * jnp.float32(0.5), m)
    e = jnp.where(big, e + 1, e)
    z = (m - jnp.float32(1.0)) / (m + jnp.float32(1.0))
    z2 = z * z
    p = jnp.float32(2.0) + z2 * (
        jnp.float32(2.0 / 3.0)
        + z2 * (jnp.float32(2.0 / 5.0) + z2 * (jnp.float32(2.0 / 7.0) + z2 * jnp.float32(2.0 / 9.0)))
    )
    return z * p + e.astype(jnp.float32) * jnp.float32(LN2)


def _make_sc_loss(rows, k):
    nw = NC * NS
    rpw = rows // nw
    ch = 256
    nch = rpw // ch
    gpc = ch // L

    mesh = plsc.VectorSubcoreMesh(core_axis_name="c", subcore_axis_name="s")

    @functools.partial(
        pl.kernel,
        out_type=(
            jax.ShapeDtypeStruct((nw, L), jnp.float32),
            jax.ShapeDtypeStruct((nw, L), jnp.float32),
        ),
        mesh=mesh,
        compiler_params=pltpu.CompilerParams(needs_layout_passes=False),
        scratch_types=[
            pltpu.VMEM((ch * k,), jnp.float32),
            pltpu.VMEM((ch,), jnp.int32),
            pltpu.VMEM((ch,), jnp.int32),
            pltpu.VMEM((k,), jnp.float32),
            pltpu.VMEM((L,), jnp.float32),
            pltpu.VMEM((L,), jnp.float32),
        ],
    )
    def sc_loss(x_hbm, gt_hbm, keep_hbm, w_hbm, wnll_hbm, wsum_hbm,
                xbuf, gtbuf, kbuf, wbuf, st_wnll, st_wsum):
        wid = lax.axis_index("s") * NC + lax.axis_index("c")
        base = wid * rpw
        pltpu.sync_copy(w_hbm, wbuf)
        lanes = lax.iota(jnp.int32, L)
        zeros = jnp.zeros((L,), jnp.float32)

        def chunk_body(ci, carry):
            a_wnll, a_w = carry
            start = base + ci * ch
            pltpu.sync_copy(x_hbm.at[pl.ds(start * k, ch * k)], xbuf)
            pltpu.sync_copy(gt_hbm.at[pl.ds(start, ch)], gtbuf)
            pltpu.sync_copy(keep_hbm.at[pl.ds(start, ch)], kbuf)

            def group_body(g, carry2):
                a1, a2 = carry2
                flat16 = (g * L + lanes) * k
                ms = [jnp.full((L,), -1e30, jnp.float32) for _ in range(4)]
                for kk in range(k):
                    v = plsc.load_gather(xbuf, [flat16 + kk])
                    ms[kk % 4] = jnp.maximum(ms[kk % 4], v)
                m = jnp.maximum(jnp.maximum(ms[0], ms[1]), jnp.maximum(ms[2], ms[3]))
                ss = [zeros for _ in range(4)]
                for kk in range(k):
                    v = plsc.load_gather(xbuf, [flat16 + kk])
                    ss[kk % 4] = ss[kk % 4] + jnp.exp(v - m)
                s = (ss[0] + ss[1]) + (ss[2] + ss[3])
                gt16 = gtbuf[pl.ds(g * L, L)]
                xg = plsc.load_gather(xbuf, [flat16 + gt16])
                wv = plsc.load_gather(wbuf, [gt16])
                kp = kbuf[pl.ds(g * L, L)]
                w = jnp.where(kp != 0, wv, jnp.float32(0.0))
                nll = _log_1_to_k(s) + m - xg
                return (a1 + w * nll, a2 + w)

            return lax.fori_loop(0, gpc, group_body, (a_wnll, a_w))

        a_wnll, a_w = lax.fori_loop(0, nch, chunk_body, (zeros, zeros))
        st_wnll[...] = a_wnll
        st_wsum[...] = a_w
        pltpu.sync_copy(st_wnll, wnll_hbm.at[wid])
        pltpu.sync_copy(st_wsum, wsum_hbm.at[wid])

    return sc_loss


def kernel(outputs, lables, masks, weight):
    b, m, k = outputs.shape
    x = outputs.reshape(b * m * k)
    gt = lables[:, 1:].reshape(b * m)
    keep = masks[:, 1:].reshape(b * m)
    wnll, wsum = _make_sc_loss(b * m, k)(x, gt, keep, weight)
    return jnp.sum(wnll) / jnp.sum(wsum)


# speedup vs baseline: 17.9268x; 3.6385x over previous
import functools

import jax
import jax.numpy as jnp
from jax import lax
from jax.experimental import pallas as pl
from jax.experimental.pallas import tpu as pltpu
from jax.experimental.pallas import tpu_sc as plsc

NC = 2
NS = 16
L = 16

LN2 = 0.6931471805599453
SQRT2 = 1.4142135623730951


def _log_1_to_k(s):
    bits = plsc.bitcast(s, jnp.int32)
    e = (bits >> 23) - 127
    mbits = (bits & jnp.int32(0x007FFFFF)) | jnp.int32(0x3F800000)
    m = plsc.bitcast(mbits, jnp.float32)
    big = m > jnp.float32(SQRT2)
    m = jnp.where(big, m * jnp.float32(0.5), m)
    e = jnp.where(big, e + 1, e)
    z = (m - jnp.float32(1.0)) / (m + jnp.float32(1.0))
    z2 = z * z
    p = jnp.float32(2.0) + z2 * (
        jnp.float32(2.0 / 3.0)
        + z2 * (jnp.float32(2.0 / 5.0) + z2 * (jnp.float32(2.0 / 7.0) + z2 * jnp.float32(2.0 / 9.0)))
    )
    return z * p + e.astype(jnp.float32) * jnp.float32(LN2)


def _make_sc_loss(rows, k):
    nw = NC * NS
    rpw = rows // nw
    ch = 256
    nch = rpw // ch
    gpc = ch // L
    kv = k // L

    mesh = plsc.VectorSubcoreMesh(core_axis_name="c", subcore_axis_name="s")

    @functools.partial(
        pl.kernel,
        out_type=(
            jax.ShapeDtypeStruct((nw, L), jnp.float32),
            jax.ShapeDtypeStruct((nw, L), jnp.float32),
        ),
        mesh=mesh,
        compiler_params=pltpu.CompilerParams(needs_layout_passes=False),
        scratch_types=[
            pltpu.VMEM((ch * k,), jnp.float32),
            pltpu.VMEM((ch * k,), jnp.float32),
            pltpu.VMEM((ch,), jnp.int32),
            pltpu.VMEM((ch,), jnp.int32),
            pltpu.VMEM((ch,), jnp.int32),
            pltpu.VMEM((ch,), jnp.int32),
            pltpu.VMEM((k,), jnp.float32),
            pltpu.VMEM((ch * 17,), jnp.float32),
            pltpu.VMEM((ch * 17,), jnp.float32),
            pltpu.VMEM((L,), jnp.float32),
            pltpu.VMEM((L,), jnp.float32),
            pltpu.SemaphoreType.DMA,
            pltpu.SemaphoreType.DMA,
        ],
    )
    def sc_loss(x_hbm, gt_hbm, keep_hbm, w_hbm, wnll_hbm, wsum_hbm,
                xb_a, xb_b, gt_a, gt_b, kp_a, kp_b, wbuf,
                st_m, st_s, st_wnll, st_wsum, sem_a, sem_b):
        wid = lax.axis_index("s") * NC + lax.axis_index("c")
        base = wid * rpw
        pltpu.sync_copy(w_hbm, wbuf)
        lanes = lax.iota(jnp.int32, L)
        zeros = jnp.zeros((L,), jnp.float32)

        def dma_start(ci, xb, gb, kb, sem):
            st = base + ci * ch
            pltpu.make_async_copy(x_hbm.at[pl.ds(st * k, ch * k)], xb, sem).start()
            pltpu.make_async_copy(gt_hbm.at[pl.ds(st, ch)], gb, sem).start()
            pltpu.make_async_copy(keep_hbm.at[pl.ds(st, ch)], kb, sem).start()

        def dma_wait(xb, gb, kb, sem):
            pltpu.make_async_copy(x_hbm.at[pl.ds(0, ch * k)], xb, sem).wait()
            pltpu.make_async_copy(gt_hbm.at[pl.ds(0, ch)], gb, sem).wait()
            pltpu.make_async_copy(keep_hbm.at[pl.ds(0, ch)], kb, sem).wait()

        def compute_chunk(xb, gb, kb, carry):
            @plsc.parallel_loop(0, ch, 1, unroll=4)
            def _(r):
                rbase = r * k
                vs = [xb[pl.ds(rbase + j * L, L)] for j in range(kv)]
                t = vs
                while len(t) > 1:
                    t = [jnp.maximum(t[2 * i], t[2 * i + 1]) for i in range(len(t) // 2)]
                bm = jnp.max(t[0]) + zeros
                es = [jnp.exp(v - bm) for v in vs]
                while len(es) > 1:
                    es = [es[2 * i] + es[2 * i + 1] for i in range(len(es) // 2)]
                bs = jnp.sum(es[0]) + zeros
                st_m[pl.ds(r * 17, L)] = bm
                st_s[pl.ds(r * 17, L)] = bs

            def group_body(g, carry2):
                a1, a2 = carry2
                rows16 = g * L + lanes
                gt16 = gb[pl.ds(g * L, L)]
                xg = plsc.load_gather(xb, [rows16 * k + gt16])
                wv = plsc.load_gather(wbuf, [gt16])
                mv = plsc.load_gather(st_m, [rows16 * 17])
                sv = plsc.load_gather(st_s, [rows16 * 17])
                kp = kb[pl.ds(g * L, L)]
                w = jnp.where(kp != 0, wv, jnp.float32(0.0))
                nll = _log_1_to_k(sv) + mv - xg
                return (a1 + w * nll, a2 + w)

            return lax.fori_loop(0, gpc, group_body, carry)

        dma_start(0, xb_a, gt_a, kp_a, sem_a)
        dma_start(1, xb_b, gt_b, kp_b, sem_b)

        def pair_body(i, carry):
            ci = 2 * i
            dma_wait(xb_a, gt_a, kp_a, sem_a)
            carry = compute_chunk(xb_a, gt_a, kp_a, carry)

            @pl.when(ci + 2 < nch)
            def _():
                dma_start(ci + 2, xb_a, gt_a, kp_a, sem_a)

            dma_wait(xb_b, gt_b, kp_b, sem_b)
            carry = compute_chunk(xb_b, gt_b, kp_b, carry)

            @pl.when(ci + 3 < nch)
            def _():
                dma_start(ci + 3, xb_b, gt_b, kp_b, sem_b)

            return carry

        a_wnll, a_w = lax.fori_loop(0, nch // 2, pair_body, (zeros, zeros))
        st_wnll[...] = a_wnll
        st_wsum[...] = a_w
        pltpu.sync_copy(st_wnll, wnll_hbm.at[wid])
        pltpu.sync_copy(st_wsum, wsum_hbm.at[wid])

    return sc_loss


def kernel(outputs, lables, masks, weight):
    b, m, k = outputs.shape
    x = outputs.reshape(b * m * k)
    gt = lables[:, 1:].reshape(b * m)
    keep = masks[:, 1:].reshape(b * m)
    wnll, wsum = _make_sc_loss(b * m, k)(x, gt, keep, weight)
    return jnp.sum(wnll) / jnp.sum(wsum)


# speedup vs baseline: 17.9634x; 1.0020x over previous
import functools

import jax
import jax.numpy as jnp
from jax import lax
from jax.experimental import pallas as pl
from jax.experimental.pallas import tpu as pltpu
from jax.experimental.pallas import tpu_sc as plsc

NC = 2
NS = 16
L = 16

LN2 = 0.6931471805599453
SQRT2 = 1.4142135623730951


def _log_1_to_k(s):
    bits = plsc.bitcast(s, jnp.int32)
    e = (bits >> 23) - 127
    mbits = (bits & jnp.int32(0x007FFFFF)) | jnp.int32(0x3F800000)
    m = plsc.bitcast(mbits, jnp.float32)
    big = m > jnp.float32(SQRT2)
    m = jnp.where(big, m * jnp.float32(0.5), m)
    e = jnp.where(big, e + 1, e)
    z = (m - jnp.float32(1.0)) / (m + jnp.float32(1.0))
    z2 = z * z
    p = jnp.float32(2.0) + z2 * (
        jnp.float32(2.0 / 3.0)
        + z2 * (jnp.float32(2.0 / 5.0) + z2 * (jnp.float32(2.0 / 7.0) + z2 * jnp.float32(2.0 / 9.0)))
    )
    return z * p + e.astype(jnp.float32) * jnp.float32(LN2)


def _make_sc_loss(bsz, m, k):
    rows = bsz * m
    nw = NC * NS
    rpw = rows // nw
    ch = 256
    nch = rpw // ch
    gpc = ch // L
    kv = k // L

    mesh = plsc.VectorSubcoreMesh(core_axis_name="c", subcore_axis_name="s")

    @functools.partial(
        pl.kernel,
        out_type=(
            jax.ShapeDtypeStruct((nw, L), jnp.float32),
            jax.ShapeDtypeStruct((nw, L), jnp.float32),
        ),
        mesh=mesh,
        compiler_params=pltpu.CompilerParams(needs_layout_passes=False),
        scratch_types=[
            pltpu.VMEM((ch * k,), jnp.float32),
            pltpu.VMEM((ch * k,), jnp.float32),
            pltpu.VMEM((ch + 8,), jnp.int32),
            pltpu.VMEM((ch + 8,), jnp.int32),
            pltpu.VMEM((ch + 8,), jnp.int32),
            pltpu.VMEM((ch + 8,), jnp.int32),
            pltpu.VMEM((k,), jnp.float32),
            pltpu.VMEM((ch * 17,), jnp.float32),
            pltpu.VMEM((ch * 17,), jnp.float32),
            pltpu.VMEM((L,), jnp.float32),
            pltpu.VMEM((L,), jnp.float32),
            pltpu.SemaphoreType.DMA,
            pltpu.SemaphoreType.DMA,
        ],
    )
    def sc_loss(x_hbm, lab_hbm, msk_hbm, w_hbm, wnll_hbm, wsum_hbm,
                xb_a, xb_b, gt_a, gt_b, kp_a, kp_b, wbuf,
                st_m, st_s, st_wnll, st_wsum, sem_a, sem_b):
        wid = lax.axis_index("s") * NC + lax.axis_index("c")
        base = wid * rpw
        bb = base // m
        pltpu.sync_copy(w_hbm, wbuf)
        lanes = lax.iota(jnp.int32, L)
        zeros = jnp.zeros((L,), jnp.float32)

        def dma_start(ci, xb, gb, kb, sem):
            st = base + ci * ch
            off = st + bb + 1
            al = pl.multiple_of((off // 8) * 8, 8)
            pltpu.make_async_copy(x_hbm.at[pl.ds(st * k, ch * k)], xb, sem).start()
            pltpu.make_async_copy(lab_hbm.at[pl.ds(al, ch + 8)], gb, sem).start()
            pltpu.make_async_copy(msk_hbm.at[pl.ds(al, ch + 8)], kb, sem).start()

        def dma_wait(xb, gb, kb, sem):
            pltpu.make_async_copy(x_hbm.at[pl.ds(0, ch * k)], xb, sem).wait()
            pltpu.make_async_copy(lab_hbm.at[pl.ds(0, ch + 8)], gb, sem).wait()
            pltpu.make_async_copy(msk_hbm.at[pl.ds(0, ch + 8)], kb, sem).wait()

        def compute_chunk(ci, xb, gb, kb, carry):
            shift = (base + ci * ch + bb + 1) % 8
            @plsc.parallel_loop(0, ch, 1, unroll=4)
            def _(r):
                rbase = r * k
                vs = [xb[pl.ds(rbase + j * L, L)] for j in range(kv)]
                t = vs
                while len(t) > 1:
                    t = [jnp.maximum(t[2 * i], t[2 * i + 1]) for i in range(len(t) // 2)]
                bm = jnp.max(t[0]) + zeros
                es = [jnp.exp(v - bm) for v in vs]
                while len(es) > 1:
                    es = [es[2 * i] + es[2 * i + 1] for i in range(len(es) // 2)]
                bs = jnp.sum(es[0]) + zeros
                st_m[pl.ds(r * 17, L)] = bm
                st_s[pl.ds(r * 17, L)] = bs

            def group_body(g, carry2):
                a1, a2 = carry2
                rows16 = g * L + lanes
                gt16 = gb[pl.ds(shift + g * L, L)]
                xg = plsc.load_gather(xb, [rows16 * k + gt16])
                wv = plsc.load_gather(wbuf, [gt16])
                mv = plsc.load_gather(st_m, [rows16 * 17])
                sv = plsc.load_gather(st_s, [rows16 * 17])
                kp = kb[pl.ds(shift + g * L, L)]
                w = jnp.where(kp != 0, wv, jnp.float32(0.0))
                nll = _log_1_to_k(sv) + mv - xg
                return (a1 + w * nll, a2 + w)

            return lax.fori_loop(0, gpc, group_body, carry)

        dma_start(0, xb_a, gt_a, kp_a, sem_a)
        dma_start(1, xb_b, gt_b, kp_b, sem_b)

        def pair_body(i, carry):
            ci = 2 * i
            dma_wait(xb_a, gt_a, kp_a, sem_a)
            carry = compute_chunk(ci, xb_a, gt_a, kp_a, carry)

            @pl.when(ci + 2 < nch)
            def _():
                dma_start(ci + 2, xb_a, gt_a, kp_a, sem_a)

            dma_wait(xb_b, gt_b, kp_b, sem_b)
            carry = compute_chunk(ci + 1, xb_b, gt_b, kp_b, carry)

            @pl.when(ci + 3 < nch)
            def _():
                dma_start(ci + 3, xb_b, gt_b, kp_b, sem_b)

            return carry

        a_wnll, a_w = lax.fori_loop(0, nch // 2, pair_body, (zeros, zeros))
        st_wnll[...] = a_wnll
        st_wsum[...] = a_w
        pltpu.sync_copy(st_wnll, wnll_hbm.at[wid])
        pltpu.sync_copy(st_wsum, wsum_hbm.at[wid])

    return sc_loss


def kernel(outputs, lables, masks, weight):
    b, m, k = outputs.shape
    x = outputs.reshape(b * m * k)
    lab = lables.reshape(b * (m + 1))
    msk = masks.reshape(b * (m + 1))
    wnll, wsum = _make_sc_loss(b, m, k)(x, lab, msk, weight)
    return jnp.sum(wnll) / jnp.sum(wsum)


# speedup vs baseline: 18.1062x; 1.0079x over previous
import functools

import jax
import jax.numpy as jnp
from jax import lax
from jax.experimental import pallas as pl
from jax.experimental.pallas import tpu as pltpu
from jax.experimental.pallas import tpu_sc as plsc

NC = 2
NS = 16
L = 16

LN2 = 0.6931471805599453
SQRT2 = 1.4142135623730951


def _log_1_to_k(s):
    bits = plsc.bitcast(s, jnp.int32)
    e = (bits >> 23) - 127
    mbits = (bits & jnp.int32(0x007FFFFF)) | jnp.int32(0x3F800000)
    m = plsc.bitcast(mbits, jnp.float32)
    big = m > jnp.float32(SQRT2)
    m = jnp.where(big, m * jnp.float32(0.5), m)
    e = jnp.where(big, e + 1, e)
    z = (m - jnp.float32(1.0)) / (m + jnp.float32(1.0))
    z2 = z * z
    p = jnp.float32(2.0) + z2 * (
        jnp.float32(2.0 / 3.0)
        + z2 * (jnp.float32(2.0 / 5.0) + z2 * (jnp.float32(2.0 / 7.0) + z2 * jnp.float32(2.0 / 9.0)))
    )
    return z * p + e.astype(jnp.float32) * jnp.float32(LN2)


def _make_sc_loss(bsz, m, k):
    rows = bsz * m
    nw = NC * NS
    rpw = rows // nw
    ch = 256
    nch = rpw // ch
    gpc = ch // L
    kv = k // L

    mesh = plsc.VectorSubcoreMesh(core_axis_name="c", subcore_axis_name="s")

    @functools.partial(
        pl.kernel,
        out_type=(
            jax.ShapeDtypeStruct((nw, L), jnp.float32),
            jax.ShapeDtypeStruct((nw, L), jnp.float32),
        ),
        mesh=mesh,
        compiler_params=pltpu.CompilerParams(needs_layout_passes=False),
        scratch_types=[
            pltpu.VMEM((2 * ch * k,), jnp.float32),
            pltpu.VMEM((2 * (ch + 8),), jnp.int32),
            pltpu.VMEM((2 * (ch + 8),), jnp.int32),
            pltpu.VMEM((k,), jnp.float32),
            pltpu.VMEM((ch * 17,), jnp.float32),
            pltpu.VMEM((ch * 17,), jnp.float32),
            pltpu.VMEM((L,), jnp.float32),
            pltpu.VMEM((L,), jnp.float32),
            pltpu.SemaphoreType.DMA,
            pltpu.SemaphoreType.DMA,
        ],
    )
    def sc_loss(x_hbm, lab_hbm, msk_hbm, w_hbm, wnll_hbm, wsum_hbm,
                xbuf, gtbuf, kpbuf, wbuf,
                st_m, st_s, st_wnll, st_wsum, sem_a, sem_b):
        wid = lax.axis_index("s") * NC + lax.axis_index("c")
        base = wid * rpw
        bb = base // m
        pltpu.sync_copy(w_hbm, wbuf)
        lanes = lax.iota(jnp.int32, L)
        zeros = jnp.zeros((L,), jnp.float32)
        sems = (sem_a, sem_b)

        def dma_start(ci, slot, sem):
            st = base + ci * ch
            off = st + bb + 1
            al = pl.multiple_of((off // 8) * 8, 8)
            pltpu.make_async_copy(
                x_hbm.at[pl.ds(st * k, ch * k)],
                xbuf.at[pl.ds(slot * (ch * k), ch * k)], sem).start()
            pltpu.make_async_copy(
                lab_hbm.at[pl.ds(al, ch + 8)],
                gtbuf.at[pl.ds(slot * (ch + 8), ch + 8)], sem).start()
            pltpu.make_async_copy(
                msk_hbm.at[pl.ds(al, ch + 8)],
                kpbuf.at[pl.ds(slot * (ch + 8), ch + 8)], sem).start()

        def dma_wait(slot, sem):
            pltpu.make_async_copy(
                x_hbm.at[pl.ds(0, ch * k)],
                xbuf.at[pl.ds(slot * (ch * k), ch * k)], sem).wait()
            pltpu.make_async_copy(
                lab_hbm.at[pl.ds(0, ch + 8)],
                gtbuf.at[pl.ds(slot * (ch + 8), ch + 8)], sem).wait()
            pltpu.make_async_copy(
                msk_hbm.at[pl.ds(0, ch + 8)],
                kpbuf.at[pl.ds(slot * (ch + 8), ch + 8)], sem).wait()

        def compute_chunk(ci, slot, carry):
            shift = (base + ci * ch + bb + 1) % 8
            xoff = slot * (ch * k)
            goff = slot * (ch + 8)
            @plsc.parallel_loop(0, ch, 1, unroll=4)
            def _(r):
                rbase = xoff + r * k
                vs = [xbuf[pl.ds(rbase + j * L, L)] for j in range(kv)]
                t = vs
                while len(t) > 1:
                    t = [jnp.maximum(t[2 * i], t[2 * i + 1]) for i in range(len(t) // 2)]
                bm = jnp.max(t[0]) + zeros
                es = [jnp.exp(v - bm) for v in vs]
                while len(es) > 1:
                    es = [es[2 * i] + es[2 * i + 1] for i in range(len(es) // 2)]
                bs = jnp.sum(es[0]) + zeros
                st_m[pl.ds(r * 17, L)] = bm
                st_s[pl.ds(r * 17, L)] = bs

            def group_body(g, carry2):
                a1, a2 = carry2
                rows16 = g * L + lanes
                gt16 = gtbuf[pl.ds(goff + shift + g * L, L)]
                xg = plsc.load_gather(xbuf, [xoff + rows16 * k + gt16])
                wv = plsc.load_gather(wbuf, [gt16])
                mv = plsc.load_gather(st_m, [rows16 * 17])
                sv = plsc.load_gather(st_s, [rows16 * 17])
                kp = kpbuf[pl.ds(goff + shift + g * L, L)]
                w = jnp.where(kp != 0, wv, jnp.float32(0.0))
                nll = _log_1_to_k(sv) + mv - xg
                return (a1 + w * nll, a2 + w)

            return lax.fori_loop(0, gpc, group_body, carry)

        def dma_start_d(ci, slot):
            @pl.when(slot == 0)
            def _():
                dma_start(ci, 0, sem_a)

            @pl.when(slot != 0)
            def _():
                dma_start(ci, 1, sem_b)

        def dma_wait_d(slot):
            @pl.when(slot == 0)
            def _():
                dma_wait(0, sem_a)

            @pl.when(slot != 0)
            def _():
                dma_wait(1, sem_b)

        dma_start(0, 0, sem_a)
        dma_start(1, 1, sem_b)

        def chunk_loop(ci, carry):
            slot = ci % 2
            dma_wait_d(slot)
            carry = compute_chunk(ci, slot, carry)

            @pl.when(ci + 2 < nch)
            def _():
                dma_start_d(ci + 2, slot)

            return carry

        a_wnll, a_w = lax.fori_loop(0, nch, chunk_loop, (zeros, zeros))
        st_wnll[...] = a_wnll
        st_wsum[...] = a_w
        pltpu.sync_copy(st_wnll, wnll_hbm.at[wid])
        pltpu.sync_copy(st_wsum, wsum_hbm.at[wid])

    return sc_loss


def kernel(outputs, lables, masks, weight):
    b, m, k = outputs.shape
    x = outputs.reshape(b * m * k)
    lab = lables.reshape(b * (m + 1))
    msk = masks.reshape(b * (m + 1))
    wnll, wsum = _make_sc_loss(b, m, k)(x, lab, msk, weight)
    return jnp.sum(wnll) / jnp.sum(wsum)
